# baseline (speedup 1.0000x reference)
import functools

import jax
import jax.numpy as jnp
from jax import lax
from jax.experimental import pallas as pl
from jax.experimental.pallas import tpu as pltpu
from jax.experimental.pallas import tpu_sc as plsc

_INPUT_DIM = 2048
_NUM_EXPERTS = 64
_N_TOKENS = 16384
_N_CHUNKS = 4
_CTOK = _N_TOKENS // _N_CHUNKS
_TBLK = 2048

_NW = 32
_TPW = _CTOK // _NW
_LANES = 16


def _logits_kernel(x_ref, w_ref, out_ref):
    out_ref[...] = jax.lax.dot_general(
        w_ref[...], x_ref[...],
        dimension_numbers=(((1,), (1,)), ((), ())),
        preferred_element_type=jnp.float32,
    )


def _logits_t(x_chunk, W):
    return pl.pallas_call(
        _logits_kernel,
        grid=(_CTOK // _TBLK,),
        in_specs=[
            pl.BlockSpec((_TBLK, _INPUT_DIM), lambda i: (i, 0)),
            pl.BlockSpec((_NUM_EXPERTS, _INPUT_DIM), lambda i: (0, 0)),
        ],
        out_specs=pl.BlockSpec((_NUM_EXPERTS, _TBLK), lambda i: (0, i)),
        out_shape=jax.ShapeDtypeStruct((_NUM_EXPERTS, _CTOK), jnp.float32),
        compiler_params=pltpu.CompilerParams(
            dimension_semantics=("arbitrary",),
        ),
    )(x_chunk, W)


@functools.partial(
    pl.kernel,
    out_type=[
        jax.ShapeDtypeStruct((_CTOK,), jnp.int32),
        jax.ShapeDtypeStruct((_CTOK,), jnp.int32),
        jax.ShapeDtypeStruct((_CTOK,), jnp.float32),
        jax.ShapeDtypeStruct((_CTOK,), jnp.float32),
    ],
    mesh=plsc.VectorSubcoreMesh(core_axis_name="c", subcore_axis_name="s"),
    scratch_types=[
        pltpu.VMEM((_NUM_EXPERTS, _TPW), jnp.float32),
        pltpu.VMEM((_TPW,), jnp.int32),
        pltpu.VMEM((_TPW,), jnp.int32),
        pltpu.VMEM((_TPW,), jnp.float32),
        pltpu.VMEM((_TPW,), jnp.float32),
    ],
)
def _sc_top2(logits_hbm, i1_hbm, i2_hbm, v1_hbm, v2_hbm,
             slab, i1_v, i2_v, v1_v, v2_v):
    wid = lax.axis_index("s") * 2 + lax.axis_index("c")
    base = wid * _TPW
    pltpu.sync_copy(logits_hbm.at[:, pl.ds(base, _TPW)], slab)

    def chunk_body(c, carry):
        t0 = c * _LANES
        m1 = slab[0, pl.ds(t0, _LANES)]
        i1 = jnp.zeros((_LANES,), jnp.int32)
        m2 = jnp.full((_LANES,), -jnp.inf, jnp.float32)
        i2 = jnp.zeros((_LANES,), jnp.int32)
        for e in range(1, _NUM_EXPERTS):
            l = slab[e, pl.ds(t0, _LANES)]
            gt1 = l > m1
            gt2 = l > m2
            ei = jnp.full((_LANES,), e, jnp.int32)
            i2 = jnp.where(gt1, i1, jnp.where(gt2, ei, i2))
            m2 = jnp.where(gt1, m1, jnp.where(gt2, l, m2))
            i1 = jnp.where(gt1, ei, i1)
            m1 = jnp.where(gt1, l, m1)
        e2 = jnp.exp(m2 - m1)
        s = 1.0 + e2
        i1_v[pl.ds(t0, _LANES)] = i1
        i2_v[pl.ds(t0, _LANES)] = i2
        v1_v[pl.ds(t0, _LANES)] = 1.0 / s
        v2_v[pl.ds(t0, _LANES)] = e2 / s
        return carry

    lax.fori_loop(0, _TPW // _LANES, chunk_body, 0)
    pltpu.sync_copy(i1_v, i1_hbm.at[pl.ds(base, _TPW)])
    pltpu.sync_copy(i2_v, i2_hbm.at[pl.ds(base, _TPW)])
    pltpu.sync_copy(v1_v, v1_hbm.at[pl.ds(base, _TPW)])
    pltpu.sync_copy(v2_v, v2_hbm.at[pl.ds(base, _TPW)])


def kernel(x, W):
    i1s, i2s, v1s, v2s = [], [], [], []
    for c in range(_N_CHUNKS):
        logits_t = _logits_t(lax.slice_in_dim(x, c * _CTOK, (c + 1) * _CTOK), W)
        i1, i2, v1, v2 = _sc_top2(logits_t)
        i1s.append(i1); i2s.append(i2); v1s.append(v1); v2s.append(v2)
    i1 = jnp.concatenate(i1s)
    i2 = jnp.concatenate(i2s)
    v1 = jnp.concatenate(v1s)
    v2 = jnp.concatenate(v2s)
    idx = jnp.concatenate([i1[:, None], i2[:, None]], axis=1)
    val = jnp.concatenate([v1[:, None], v2[:, None]], axis=1)
    return (idx, val)


# speedup vs baseline: 2.1032x; 2.1032x over previous
import jax
import jax.numpy as jnp
from jax.experimental import pallas as pl
from jax.experimental.pallas import tpu as pltpu

_INPUT_DIM = 2048
_NUM_EXPERTS = 64
_NB = 8
_R = 512


def _router_kernel(x_hbm, w_ref, idx_ref, val_ref, bufs, sems):
    n_tiles = x_hbm.shape[0] // _R

    def slab_copy(i, b):
        return pltpu.make_async_copy(
            x_hbm.at[pl.ds(i * _R, _R), :], bufs.at[b], sems.at[b])

    for i in range(min(_NB, n_tiles)):
        slab_copy(i, i).start()

    cols = jax.lax.broadcasted_iota(jnp.int32, (_R, _NUM_EXPERTS), 1)
    big_i = jnp.int32(_NUM_EXPERTS)
    neg = jnp.float32(-jnp.inf)

    for i in range(n_tiles):
        b = i % _NB
        slab_copy(i, b).wait()
        logits = jax.lax.dot_general(
            bufs[b], w_ref[...],
            dimension_numbers=(((1,), (1,)), ((), ())),
            preferred_element_type=jnp.float32,
        )

        m1 = jnp.max(logits, axis=1, keepdims=True)
        i1 = jnp.min(jnp.where(logits == m1, cols, big_i), axis=1, keepdims=True)
        masked = jnp.where(cols == i1, neg, logits)
        m2 = jnp.max(masked, axis=1, keepdims=True)
        i2 = jnp.min(jnp.where(masked == m2, cols, big_i), axis=1, keepdims=True)

        e = jnp.exp(m2 - m1)
        s = 1.0 + e
        idx_ref[pl.ds(i * _R, _R), :] = jnp.concatenate([i1, i2], axis=1)
        val_ref[pl.ds(i * _R, _R), :] = jnp.concatenate([1.0 / s, e / s], axis=1)

        nxt = i + _NB
        if nxt < n_tiles:
            slab_copy(nxt, b).start()


def kernel(x, W):
    n_tokens = x.shape[0]
    idx, val = pl.pallas_call(
        _router_kernel,
        in_specs=[
            pl.BlockSpec(memory_space=pl.ANY),
            pl.BlockSpec((_NUM_EXPERTS, _INPUT_DIM), lambda: (0, 0)),
        ],
        out_specs=[
            pl.BlockSpec((n_tokens, 2), lambda: (0, 0)),
            pl.BlockSpec((n_tokens, 2), lambda: (0, 0)),
        ],
        out_shape=[
            jax.ShapeDtypeStruct((n_tokens, 2), jnp.int32),
            jax.ShapeDtypeStruct((n_tokens, 2), jnp.float32),
        ],
        scratch_shapes=[
            pltpu.VMEM((_NB, _R, _INPUT_DIM), jnp.float32),
            pltpu.SemaphoreType.DMA((_NB,)),
        ],
    )(x, W)
    return (idx, val)


# speedup vs baseline: 2.7066x; 1.2869x over previous
import jax
import jax.numpy as jnp
from jax.experimental import pallas as pl
from jax.experimental.pallas import tpu as pltpu

_INPUT_DIM = 2048
_NUM_EXPERTS = 64
_TBLK = 2048


def _to_ordered(b):
    return b ^ ((b >> 31) & jnp.int32(0x7FFFFFFF))


def _router_kernel(x_ref, w_ref, idx_ref, val_ref):
    logits = jax.lax.dot_general(
        x_ref[...], w_ref[...],
        dimension_numbers=(((1,), (1,)), ((), ())),
        preferred_element_type=jnp.float32,
    )

    rcols = jax.lax.broadcasted_iota(jnp.int32, logits.shape, 1)
    rcols = jnp.int32(_NUM_EXPERTS - 1) - rcols

    b = jax.lax.bitcast_convert_type(logits, jnp.int32)
    key = (_to_ordered(b) & jnp.int32(~63)) | rcols

    k1 = jnp.max(key, axis=1, keepdims=True)
    masked = jnp.where(key == k1, jnp.int32(-(2**31)), key)
    k2 = jnp.max(masked, axis=1, keepdims=True)

    i1 = jnp.int32(_NUM_EXPERTS - 1) - (k1 & 63)
    i2 = jnp.int32(_NUM_EXPERTS - 1) - (k2 & 63)
    m1 = jax.lax.bitcast_convert_type(_to_ordered(k1 & jnp.int32(~63)), jnp.float32)
    m2 = jax.lax.bitcast_convert_type(_to_ordered(k2 & jnp.int32(~63)), jnp.float32)

    e = jnp.exp(m2 - m1)
    s = 1.0 + e
    idx_ref[...] = jnp.concatenate([i1, i2], axis=1)
    val_ref[...] = jnp.concatenate([1.0 / s, e / s], axis=1)


def kernel(x, W):
    n_tokens = x.shape[0]
    grid = (n_tokens // _TBLK,)
    idx, val = pl.pallas_call(
        _router_kernel,
        grid=grid,
        in_specs=[
            pl.BlockSpec((_TBLK, _INPUT_DIM), lambda i: (i, 0)),
            pl.BlockSpec((_NUM_EXPERTS, _INPUT_DIM), lambda i: (0, 0)),
        ],
        out_specs=[
            pl.BlockSpec((_TBLK, 2), lambda i: (i, 0)),
            pl.BlockSpec((_TBLK, 2), lambda i: (i, 0)),
        ],
        out_shape=[
            jax.ShapeDtypeStruct((n_tokens, 2), jnp.int32),
            jax.ShapeDtypeStruct((n_tokens, 2), jnp.float32),
        ],
        compiler_params=pltpu.CompilerParams(
            dimension_semantics=("parallel",),
        ),
    )(x, W)
    return (idx, val)
